# baseline (speedup 1.0000x reference)
import jax
import jax.numpy as jnp
from jax.experimental import pallas as pl
from jax.experimental.pallas import tpu as pltpu

_LO = -1.03
_HI = 1.03

_BR = 256


def _mlp_kernel(x_ref, w1_ref, b1_ref, w2_ref, b2_ref, g_ref, fm_ref, o_ref):
    x = x_ref[...]
    t = jnp.where((x < _LO) | (x > _HI), fm_ref[...], 0.0)
    s = jax.lax.dot_general(t.astype(jnp.bfloat16), g_ref[...],
                            (((1,), (0,)), ((), ())),
                            preferred_element_type=jnp.float32)
    xb = x.astype(jnp.bfloat16)
    h = jax.lax.dot_general(xb, w1_ref[...], (((1,), (0,)), ((), ())),
                            preferred_element_type=jnp.float32)
    h = jnp.maximum(h + b1_ref[...], 0.0)
    v = jax.lax.dot_general(h.astype(jnp.bfloat16), w2_ref[...],
                            (((1,), (0,)), ((), ())),
                            preferred_element_type=jnp.float32)
    v = v + b2_ref[...]
    o_ref[...] = jnp.where(s > 0.5, 0.0, v)


def kernel(xt, W1, b1, W2, b2):
    n = xt.shape[0]
    x2 = xt.reshape(n // 32, 128)
    eye = jnp.eye(32, dtype=jnp.bfloat16)
    w1big = jnp.kron(eye, W1.astype(jnp.bfloat16))
    w2big = jnp.kron(eye, W2.astype(jnp.bfloat16))
    gmat = jnp.kron(eye, jnp.ones((4, 3), jnp.bfloat16))
    b1big = jnp.tile(b1, 32).reshape(1, 2048)
    b2big = jnp.tile(b2, 32).reshape(1, 96)
    fmask = jnp.tile(jnp.array([1.0, 1.0, 1.0, 0.0], jnp.float32), 32)
    fmask = fmask.reshape(1, 128)

    rows = n // 32
    grid = (rows // _BR,)
    out = pl.pallas_call(
        _mlp_kernel,
        grid=grid,
        in_specs=[
            pl.BlockSpec((_BR, 128), lambda i: (i, 0)),
            pl.BlockSpec((128, 2048), lambda i: (0, 0)),
            pl.BlockSpec((1, 2048), lambda i: (0, 0)),
            pl.BlockSpec((2048, 96), lambda i: (0, 0)),
            pl.BlockSpec((1, 96), lambda i: (0, 0)),
            pl.BlockSpec((128, 96), lambda i: (0, 0)),
            pl.BlockSpec((1, 128), lambda i: (0, 0)),
        ],
        out_specs=pl.BlockSpec((_BR, 96), lambda i: (i, 0)),
        out_shape=jax.ShapeDtypeStruct((rows, 96), xt.dtype),
        compiler_params=pltpu.CompilerParams(
            dimension_semantics=("arbitrary",),
        ),
    )(x2, w1big, b1big, w2big, b2big, gmat, fmask)
    return out.reshape(n, 3)


# speedup vs baseline: 32.7025x; 32.7025x over previous
import jax
import jax.numpy as jnp
from jax.experimental import pallas as pl
from jax.experimental.pallas import tpu as pltpu

_LO = -1.03
_HI = 1.03

_BN = 32768


def _mlp_kernel(x_ref, w1_ref, b1_ref, w2_ref, b2_ref, o_ref):
    x = x_ref[...]
    out_of = ((x[:3, :] < _LO) | (x[:3, :] > _HI)).any(axis=0, keepdims=True)
    xb = x.astype(jnp.bfloat16)
    h = jax.lax.dot_general(w1_ref[...], xb, (((1,), (0,)), ((), ())),
                            preferred_element_type=jnp.float32)
    h = jnp.maximum(h + b1_ref[...], 0.0).astype(jnp.bfloat16)
    v = jax.lax.dot_general(w2_ref[...], h, (((1,), (0,)), ((), ())),
                            preferred_element_type=jnp.float32)
    v = v + b2_ref[...]
    o_ref[...] = jnp.where(out_of, 0.0, v)


def kernel(xt, W1, b1, W2, b2):
    n = xt.shape[0]
    x_t = xt.T
    w1t = W1.T.astype(jnp.bfloat16)
    w2t = W2.T.astype(jnp.bfloat16)
    b1t = b1.reshape(64, 1)
    b2t = b2.reshape(3, 1)

    grid = (n // _BN,)
    out_t = pl.pallas_call(
        _mlp_kernel,
        grid=grid,
        in_specs=[
            pl.BlockSpec((4, _BN), lambda i: (0, i)),
            pl.BlockSpec((64, 4), lambda i: (0, 0)),
            pl.BlockSpec((64, 1), lambda i: (0, 0)),
            pl.BlockSpec((3, 64), lambda i: (0, 0)),
            pl.BlockSpec((3, 1), lambda i: (0, 0)),
        ],
        out_specs=pl.BlockSpec((3, _BN), lambda i: (0, i)),
        out_shape=jax.ShapeDtypeStruct((3, n), xt.dtype),
        compiler_params=pltpu.CompilerParams(
            dimension_semantics=("arbitrary",),
        ),
    )(x_t, w1t, b1t, w2t, b2t)
    return out_t.T


# speedup vs baseline: 34.4334x; 1.0529x over previous
import jax
import jax.numpy as jnp
from jax.experimental import pallas as pl
from jax.experimental.pallas import tpu as pltpu

_HI = 1.03

_BN = 65536


def _mlp_kernel(x_ref, w1_ref, w2_ref, b2_ref, o_ref):
    x = x_ref[...]
    keep = (jnp.max(jnp.abs(x[:3, :]), axis=0, keepdims=True)
            <= _HI).astype(jnp.float32)
    xb = x.astype(jnp.bfloat16)
    ones = jnp.ones((1, xb.shape[1]), jnp.bfloat16)
    x5 = jnp.concatenate([xb, ones], axis=0)
    h = jax.lax.dot_general(w1_ref[...], x5, (((1,), (0,)), ((), ())),
                            preferred_element_type=jnp.float32)
    h = jnp.maximum(h.astype(jnp.bfloat16), 0)
    v = jax.lax.dot_general(w2_ref[...], h, (((1,), (0,)), ((), ())),
                            preferred_element_type=jnp.float32)
    o_ref[...] = (v + b2_ref[...]) * keep


def kernel(xt, W1, b1, W2, b2):
    n = xt.shape[0]
    x_t = xt.T
    w1a = jnp.concatenate([W1.T, b1.reshape(64, 1)], axis=1)
    w1a = w1a.astype(jnp.bfloat16)
    w2t = W2.T.astype(jnp.bfloat16)
    b2t = b2.reshape(3, 1)

    grid = (n // _BN,)
    out_t = pl.pallas_call(
        _mlp_kernel,
        grid=grid,
        in_specs=[
            pl.BlockSpec((4, _BN), lambda i: (0, i)),
            pl.BlockSpec((64, 5), lambda i: (0, 0)),
            pl.BlockSpec((3, 64), lambda i: (0, 0)),
            pl.BlockSpec((3, 1), lambda i: (0, 0)),
        ],
        out_specs=pl.BlockSpec((3, _BN), lambda i: (0, i)),
        out_shape=jax.ShapeDtypeStruct((3, n), xt.dtype),
        compiler_params=pltpu.CompilerParams(
            dimension_semantics=("arbitrary",),
        ),
    )(x_t, w1a, w2t, b2t)
    return out_t.T
